# baseline (speedup 1.0000x reference)
import functools

import jax
import jax.numpy as jnp
from jax import lax
from jax.experimental import pallas as pl
from jax.experimental.pallas import tpu as pltpu
from jax.experimental.pallas import tpu_sc as plsc

N_TASKS_K = 8
LATENT_K = 1024
OUT_K = 128
N_TOK = 8192
CH = 1024
NC = N_TOK // CH
QTR = CH // 4
SLOTS = 2

TPW = 256
NZ = 7 * TPW // 128


def _tc_body(task_hbm, x_hbm, w_hbm, bias_hbm, yc_hbm,
             wbuf, biasbuf, taskbuf, xbuf, obuf,
             sem_w, sem_bias, sem_task, sems_x, sems_o):
    def x_copy(c, h):
        slot = c % SLOTS
        return pltpu.make_async_copy(
            x_hbm.at[pl.ds(c * CH + h * QTR, QTR), :],
            xbuf.at[slot, pl.ds(h * QTR, QTR), :],
            sems_x.at[slot, h])

    def o_copy(c):
        slot = c % SLOTS
        return pltpu.make_async_copy(
            obuf.at[slot],
            yc_hbm.at[pl.ds(c * CH, CH), :],
            sems_o.at[slot])

    cw = pltpu.make_async_copy(w_hbm, wbuf, sem_w)
    cb = pltpu.make_async_copy(bias_hbm, biasbuf, sem_bias)
    ct = pltpu.make_async_copy(task_hbm, taskbuf, sem_task)
    cw.start(); cb.start(); ct.start()
    for c in range(2):
        for h in range(4):
            x_copy(c, h).start()
    cw.wait(); cb.wait(); ct.wait()
    w2d = wbuf[...].reshape(N_TASKS_K * OUT_K, LATENT_K)
    wt = jnp.transpose(w2d).astype(jnp.bfloat16)

    for c in range(NC):
        slot = c % SLOTS
        for h in range(4):
            x_copy(c, h).wait()
        if c >= 2:
            o_copy(c - 2).wait()
        xb = xbuf[slot].astype(jnp.bfloat16)
        y = jnp.dot(xb, wt, preferred_element_type=jnp.float32)
        y = y + biasbuf[...]
        tb = taskbuf[0, pl.ds(c * CH, CH)]
        acc = None
        for t in range(N_TASKS_K):
            m = (tb == t).astype(jnp.float32)[:, None]
            term = y[:, t * OUT_K:(t + 1) * OUT_K] * m
            acc = term if acc is None else acc + term
        obuf[slot] = acc
        o_copy(c).start()
        if c + 2 < NC:
            for h in range(4):
                x_copy(c + 2, h).start()
    o_copy(NC - 2).wait()
    o_copy(NC - 1).wait()


def _tc_compact(task, x, W, bias_row):
    return pl.pallas_call(
        _tc_body,
        in_specs=[pl.BlockSpec(memory_space=pl.ANY)] * 4,
        out_specs=pl.BlockSpec(memory_space=pl.ANY),
        out_shape=jax.ShapeDtypeStruct((N_TOK, OUT_K), jnp.float32),
        scratch_shapes=[
            pltpu.VMEM((N_TASKS_K, OUT_K, LATENT_K), jnp.float32),
            pltpu.VMEM((1, N_TASKS_K * OUT_K), jnp.float32),
            pltpu.VMEM((1, N_TOK), jnp.int32),
            pltpu.VMEM((SLOTS, CH, LATENT_K), jnp.float32),
            pltpu.VMEM((SLOTS, CH, OUT_K), jnp.float32),
            pltpu.SemaphoreType.DMA,
            pltpu.SemaphoreType.DMA,
            pltpu.SemaphoreType.DMA,
            pltpu.SemaphoreType.DMA((SLOTS, 4)),
            pltpu.SemaphoreType.DMA((SLOTS,)),
        ],
    )(task, x, W, bias_row)


def _sc_body(yc_hbm, task_hbm, zsrc_hbm, out_hbm,
             taskv, rowsv, zrows, didx1, didx2, zidx1, zidx2,
             sem_in, sem_sc):
    wid = lax.axis_index("s") * 2 + lax.axis_index("c")
    base = wid * TPW

    pltpu.sync_copy(task_hbm.at[pl.ds(base, TPW)], taskv)
    pltpu.sync_copy(zsrc_hbm, zrows)
    pltpu.make_async_copy(yc_hbm.at[pl.ds(base, TPW), :], rowsv, sem_in).start()

    def build_didx(g, carry):
        tv = taskv[pl.ds(g * 16, 16)]
        pos = base + g * 16 + lax.iota(jnp.int32, 16)
        didx1[pl.ds(g * 16, 16)] = tv * N_TOK + pos
        return carry
    lax.fori_loop(0, TPW // 16, build_didx, 0)

    def build_zidx(g, carry):
        tv = taskv[pl.ds(g * 16, 16)]
        pos = base + g * 16 + lax.iota(jnp.int32, 16)
        for j in range(N_TASKS_K - 1):
            tj = j + jnp.where(tv <= j, 1, 0)
            zidx1[pl.ds((g * 7 + j) * 16, 16)] = tj * N_TOK + pos
        return carry
    lax.fori_loop(0, TPW // 16, build_zidx, 0)

    for k in range(2):
        for j in range(8):
            didx2[k, pl.ds(j * 16, 16)] = didx1[pl.ds(k * 128 + j * 16, 16)]
    for k in range(NZ):
        for j in range(8):
            zidx2[k, pl.ds(j * 16, 16)] = zidx1[pl.ds(k * 128 + j * 16, 16)]

    pltpu.make_async_copy(yc_hbm.at[pl.ds(base, TPW), :], rowsv, sem_in).wait()

    scats = []
    for k in range(2):
        scats.append(pltpu.make_async_copy(
            rowsv.at[pl.ds(k * 128, 128), :],
            out_hbm.at[didx2.at[k]], sem_sc))
    for k in range(NZ):
        scats.append(pltpu.make_async_copy(
            zrows, out_hbm.at[zidx2.at[k]], sem_sc))
    for s in scats:
        s.start()
    for s in scats:
        s.wait()


_sc_scatter = functools.partial(
    pl.kernel,
    out_type=jax.ShapeDtypeStruct((N_TASKS_K * N_TOK, OUT_K), jnp.float32),
    mesh=plsc.VectorSubcoreMesh(core_axis_name="c", subcore_axis_name="s"),
    scratch_types=[
        pltpu.VMEM((TPW,), jnp.int32),
        pltpu.VMEM((TPW, OUT_K), jnp.float32),
        pltpu.VMEM((128, OUT_K), jnp.float32),
        pltpu.VMEM((TPW,), jnp.int32),
        pltpu.VMEM((2, 128), jnp.int32),
        pltpu.VMEM((NZ * 128 + 16,), jnp.int32),
        pltpu.VMEM((NZ, 128), jnp.int32),
        pltpu.SemaphoreType.DMA,
        pltpu.SemaphoreType.DMA,
    ],
)(_sc_body)


def kernel(output_latents, output_task_index, W, b):
    Bsz, T, D = output_latents.shape
    N = Bsz * T
    x = output_latents.reshape(N, D)
    task2d = output_task_index.reshape(1, N).astype(jnp.int32)
    task1d = output_task_index.reshape(N).astype(jnp.int32)
    bias_row = b.reshape(1, N_TASKS_K * OUT_K)
    zsrc = jnp.zeros((128, OUT_K), jnp.float32)

    yc = _tc_compact(task2d, x, W, bias_row)
    outflat = _sc_scatter(yc, task1d, zsrc)
    return outflat.reshape(N_TASKS_K, Bsz, T, OUT_K)


# speedup vs baseline: 2.1460x; 2.1460x over previous
import jax
import jax.numpy as jnp
from jax.experimental import pallas as pl
from jax.experimental.pallas import tpu as pltpu

N_TASKS_K = 8
LATENT_K = 1024
OUT_K = 128
CH = 1024
NC = 8192 // CH
QTR = CH // 4


SLOTS = 2


def _body(task_hbm, x_hbm, w_hbm, bias_hbm, out_hbm,
          wbuf, biasbuf, taskbuf, xbuf, obuf,
          sem_w, sem_bias, sem_task, sems_x, sems_o):
    def x_copy(c, h):
        slot = c % SLOTS
        return pltpu.make_async_copy(
            x_hbm.at[pl.ds(c * CH + h * QTR, QTR), :],
            xbuf.at[slot, pl.ds(h * QTR, QTR), :],
            sems_x.at[slot, h])

    def o_copy(c, h):
        slot = c % SLOTS
        return pltpu.make_async_copy(
            obuf.at[slot, pl.ds(h * 2, 2), :, :],
            out_hbm.at[pl.ds(h * 2, 2), pl.ds(c * CH, CH), :],
            sems_o.at[slot, h])

    cw = pltpu.make_async_copy(w_hbm, wbuf, sem_w)
    cb = pltpu.make_async_copy(bias_hbm, biasbuf, sem_bias)
    ct = pltpu.make_async_copy(task_hbm, taskbuf, sem_task)
    cw.start(); cb.start(); ct.start()
    for c in range(2):
        for h in range(4):
            x_copy(c, h).start()
    cw.wait(); cb.wait(); ct.wait()
    w2d = wbuf[...].reshape(N_TASKS_K * OUT_K, LATENT_K)
    wt = jnp.transpose(w2d).astype(jnp.bfloat16)

    for c in range(NC):
        slot = c % SLOTS
        for h in range(4):
            x_copy(c, h).wait()
        if c >= 2:
            for h in range(4):
                o_copy(c - 2, h).wait()
        xb = xbuf[slot].astype(jnp.bfloat16)
        y = jnp.dot(xb, wt, preferred_element_type=jnp.float32)
        y = y + biasbuf[...]
        tb = taskbuf[0, pl.ds(c * CH, CH)]
        for t in range(N_TASKS_K):
            m = (tb == t).astype(jnp.float32)[:, None]
            obuf[slot, t, :, :] = y[:, t * OUT_K:(t + 1) * OUT_K] * m
        for h in range(4):
            o_copy(c, h).start()
        if c + 2 < NC:
            for h in range(4):
                x_copy(c + 2, h).start()
    for h in range(4):
        o_copy(NC - 2, h).wait()
    for h in range(4):
        o_copy(NC - 1, h).wait()


def kernel(output_latents, output_task_index, W, b):
    Bsz, T, D = output_latents.shape
    N = Bsz * T
    x = output_latents.reshape(N, D)
    task = output_task_index.reshape(1, N).astype(jnp.int32)
    bias_row = b.reshape(1, N_TASKS_K * OUT_K)

    out = pl.pallas_call(
        _body,
        in_specs=[pl.BlockSpec(memory_space=pl.ANY)] * 4,
        out_specs=pl.BlockSpec(memory_space=pl.ANY),
        out_shape=jax.ShapeDtypeStruct((N_TASKS_K, N, OUT_K), jnp.float32),
        scratch_shapes=[
            pltpu.VMEM((N_TASKS_K, OUT_K, LATENT_K), jnp.float32),
            pltpu.VMEM((1, N_TASKS_K * OUT_K), jnp.float32),
            pltpu.VMEM((1, N), jnp.int32),
            pltpu.VMEM((SLOTS, CH, D), jnp.float32),
            pltpu.VMEM((SLOTS, N_TASKS_K, CH, OUT_K), jnp.float32),
            pltpu.SemaphoreType.DMA,
            pltpu.SemaphoreType.DMA,
            pltpu.SemaphoreType.DMA,
            pltpu.SemaphoreType.DMA((SLOTS, 4)),
            pltpu.SemaphoreType.DMA((SLOTS, 4)),
        ],
    )(task, x, W, bias_row)
    return out.reshape(N_TASKS_K, Bsz, T, OUT_K)
